# baseline (speedup 1.0000x reference)
import functools

import jax
import jax.numpy as jnp
from jax import lax
from jax.experimental import pallas as pl
from jax.experimental.pallas import tpu as pltpu
from jax.experimental.pallas import tpu_sc as plsc

B, C, H, W = 2, 96, 512, 512
HW = H * W
NC, NS = 2, 16
NW = NC * NS
PPW = B * HW // NW
M = 128
NCHUNK = PPW // M
OB = M
OROWS = B * C * (HW // OB)


def _warp_sc(table, dispi, dispj):
    mesh = plsc.VectorSubcoreMesh(core_axis_name="c", subcore_axis_name="s")

    @functools.partial(
        pl.kernel,
        mesh=mesh,
        compiler_params=pltpu.CompilerParams(
            needs_layout_passes=False,
            use_tc_tiling_on_sc=False,
        ),
        out_type=jax.ShapeDtypeStruct((OROWS, OB), jnp.float32),
        scratch_types=[
            pltpu.VMEM((M,), jnp.float32),
            pltpu.VMEM((M,), jnp.float32),
            pltpu.VMEM((M,), jnp.float32),
            pltpu.VMEM((M,), jnp.float32),
            pltpu.VMEM((M,), jnp.int32),
            pltpu.VMEM((M,), jnp.int32),
            pltpu.VMEM((M,), jnp.int32),
            pltpu.VMEM((M,), jnp.int32),
            pltpu.VMEM((M, C), jnp.float32),
            pltpu.VMEM((M, C), jnp.float32),
            pltpu.VMEM((M, C), jnp.float32),
            pltpu.VMEM((M, C), jnp.float32),
            pltpu.VMEM((C, M), jnp.float32),
            pltpu.VMEM((C,), jnp.int32),
            pltpu.SemaphoreType.DMA,
            pltpu.SemaphoreType.DMA,
        ],
    )
    def k(table_h, di_h, dj_h, out_h,
          di_v, dj_v, wi_v, wj_v,
          i00, i01, i10, i11,
          g00, g01, g10, g11,
          oT, oidx, gsem, ssem):
        wid = lax.axis_index("s") * NC + lax.axis_index("c")
        pix0 = wid * PPW
        b = pix0 // HW
        tab_base = b * HW

        def chunk(t, carry):
            p0 = pix0 + t * M
            pltpu.sync_copy(di_h.at[pl.ds(p0, M)], di_v)
            pltpu.sync_copy(dj_h.at[pl.ds(p0, M)], dj_v)

            for g in range(M // 16):
                pvec = p0 + g * 16 + lax.iota(jnp.int32, 16)
                ii = lax.shift_right_logical(pvec, 9) & (H - 1)
                jj = pvec & (W - 1)
                li = ii.astype(jnp.float32) + di_v[pl.ds(g * 16, 16)]
                lj = jj.astype(jnp.float32) + dj_v[pl.ds(g * 16, 16)]
                lic = jnp.clip(li, 0.0, float(H - 1))
                ljc = jnp.clip(lj, 0.0, float(W - 1))
                ai = jnp.minimum(lic.astype(jnp.int32), H - 2)
                aj = jnp.minimum(ljc.astype(jnp.int32), W - 2)
                wi_v[pl.ds(g * 16, 16)] = lic - ai.astype(jnp.float32)
                wj_v[pl.ds(g * 16, 16)] = ljc - aj.astype(jnp.float32)
                r00 = tab_base + ai * W + aj
                i00[pl.ds(g * 16, 16)] = r00
                i01[pl.ds(g * 16, 16)] = r00 + 1
                i10[pl.ds(g * 16, 16)] = r00 + W
                i11[pl.ds(g * 16, 16)] = r00 + W + 1

            c0 = pltpu.async_copy(table_h.at[i00], g00, gsem)
            c1 = pltpu.async_copy(table_h.at[i01], g01, gsem)
            c2 = pltpu.async_copy(table_h.at[i10], g10, gsem)
            c3 = pltpu.async_copy(table_h.at[i11], g11, gsem)
            c0.wait()
            c1.wait()
            c2.wait()
            c3.wait()

            lane = lax.iota(jnp.int32, 16)

            def pix(p, cc):
                pv = jnp.full((16,), p, jnp.int32)
                w_i = plsc.load_gather(wi_v, [pv])
                w_j = plsc.load_gather(wj_v, [pv])
                for q in range(C // 16):
                    sl = pl.ds(q * 16, 16)
                    v00 = g00[p, sl]
                    v01 = g01[p, sl]
                    v10 = g10[p, sl]
                    v11 = g11[p, sl]
                    top = v00 + w_j * (v01 - v00)
                    bot = v10 + w_j * (v11 - v10)
                    o = top + w_i * (bot - top)
                    plsc.store_scatter(oT, [lane + q * 16, pv], o)
                return cc

            lax.fori_loop(0, M, pix, 0)

            colblk = (p0 - tab_base) // OB
            orow0 = b * C * (HW // OB) + colblk
            for q in range(C // 16):
                oidx[pl.ds(q * 16, 16)] = (
                    orow0 + (lane + q * 16) * (HW // OB))
            pltpu.async_copy(oT, out_h.at[oidx], ssem).wait()
            return carry

        lax.fori_loop(0, NCHUNK, chunk, 0)

    return k(table, dispi, dispj)


def kernel(img, trf):
    table = jnp.transpose(img, (0, 2, 3, 1)).reshape(B * HW, C)
    dispi = trf[:, 0].reshape(B * HW)
    dispj = trf[:, 1].reshape(B * HW)
    out = _warp_sc(table, dispi, dispj)
    return out.reshape(B, C, H, W)
